# baseline (speedup 1.0000x reference)
import functools

import jax
import jax.numpy as jnp
from jax import lax
from jax.experimental import pallas as pl
from jax.experimental.pallas import tpu as pltpu
from jax.experimental.pallas import tpu_sc as plsc

NUM_ROWS = 1000000
DIM = 64
ROWS_PER_BLOCK = 8000
NUM_BLOCKS = NUM_ROWS // ROWS_PER_BLOCK

NW = 32
CHUNK = 128
NCHUNK = 104
PER_W = CHUNK * NCHUNK


def _sigma_body(w_ref, u_ref, o_ref, g_acc, t_acc):
    i = pl.program_id(0)

    @pl.when(i == 0)
    def _init():
        g_acc[...] = jnp.zeros_like(g_acc)
        t_acc[...] = jnp.zeros_like(t_acc)

    w = w_ref[...]
    u = u_ref[...].reshape(1, ROWS_PER_BLOCK)
    g_acc[...] += lax.dot_general(
        w, w, (((0,), (0,)), ((), ())), preferred_element_type=jnp.float32)
    t_acc[...] += jnp.dot(u, w, preferred_element_type=jnp.float32)

    @pl.when(i == NUM_BLOCKS - 1)
    def _fini():
        eps = 1e-12
        t = t_acc[...]
        nt = jnp.sqrt(jnp.sum(t * t))
        v = t / (nt + eps)
        gv = jnp.dot(v, g_acc[...], preferred_element_type=jnp.float32)
        s2 = jnp.sum(gv * v)
        sigma = s2 / (jnp.sqrt(s2) + eps)
        o_ref[...] = jnp.broadcast_to(1.0 / sigma, (8, 128))


def _sigma_pass(weight, u):
    return pl.pallas_call(
        _sigma_body,
        grid=(NUM_BLOCKS,),
        in_specs=[
            pl.BlockSpec((ROWS_PER_BLOCK, DIM), lambda i: (i, 0)),
            pl.BlockSpec((1, 1, ROWS_PER_BLOCK), lambda i: (i, 0, 0)),
        ],
        out_specs=pl.BlockSpec((8, 128), lambda i: (0, 0)),
        out_shape=jax.ShapeDtypeStruct((8, 128), jnp.float32),
        scratch_shapes=[
            pltpu.VMEM((DIM, DIM), jnp.float32),
            pltpu.VMEM((1, DIM), jnp.float32),
        ],
    )(weight, u.reshape(NUM_BLOCKS, 1, ROWS_PER_BLOCK))


@functools.cache
def _make_gather():
    mesh = plsc.VectorSubcoreMesh(
        core_axis_name="c", subcore_axis_name="s", num_cores=2, num_subcores=16)

    @functools.partial(
        pl.kernel,
        out_type=jax.ShapeDtypeStruct((NW * PER_W, DIM), jnp.float32),
        mesh=mesh,
        scratch_types=[
            pltpu.VMEM((NCHUNK, CHUNK), jnp.int32),
            pltpu.VMEM((16,), jnp.float32),
            pltpu.VMEM((CHUNK, DIM), jnp.float32),
            pltpu.SemaphoreType.DMA,
        ],
        compiler_params=pltpu.CompilerParams(use_tc_tiling_on_sc=False),
    )
    def _gather(table_hbm, idx_hbm, scale_hbm, out_hbm,
                idx_v, scale_v, rows_v, sem):
        wid = lax.axis_index("s") * 2 + lax.axis_index("c")
        pltpu.sync_copy(idx_hbm.at[wid], idx_v)
        pltpu.sync_copy(scale_hbm, scale_v)
        s = scale_v[...]
        base = wid * PER_W

        def chunk(j, carry):
            pltpu.async_copy(table_hbm.at[idx_v.at[j]], rows_v, sem).wait()

            def row(r, c):
                for k in range(DIM // 16):
                    rows_v[r, pl.ds(16 * k, 16)] = (
                        rows_v[r, pl.ds(16 * k, 16)] * s)
                return c

            lax.fori_loop(0, CHUNK, row, 0, unroll=4)
            pltpu.sync_copy(rows_v, out_hbm.at[pl.ds(base + j * CHUNK, CHUNK)])
            return carry

        lax.fori_loop(0, NCHUNK, chunk, 0)

    return _gather


def kernel(x, weight, u):
    inv_blk = _sigma_pass(weight, u)
    scale16 = inv_blk[0, :16]
    idx3 = x.reshape(NW, NCHUNK, CHUNK)
    out = _make_gather()(weight, idx3, scale16)
    return out.reshape(x.shape[0], x.shape[1], DIM)
